# Initial kernel scaffold; baseline (speedup 1.0000x reference)
import jax
import jax.numpy as jnp
from jax.experimental import pallas as pl


def kernel(input_x, table):
    raise NotImplementedError("write your pallas kernel here")


# speedup vs baseline: 3.3777x; 3.3777x over previous
import jax
import jax.numpy as jnp
from jax import lax
from jax.experimental import pallas as pl
from jax.experimental.pallas import tpu as pltpu
from jax.experimental.pallas import tpu_sc as plsc

B = 16384
L = 200
E = 32
V = 257

NC = 2
NS = 16
NW = NC * NS
LANES = 16

BPT = B // NW
CB = 8
ROUNDS = BPT // CB
IDX_PER_ROUND = CB * L
OUT_PER_ROUND = CB * E * L
IDX_PER_TILE = BPT * L
OUT_PER_TILE = BPT * E * L


def _body(idx_hbm, table_hbm, out_hbm, table_v, idx_v, out_v):
    wid = lax.axis_index("s") * NC + lax.axis_index("c")
    idx_base = wid * IDX_PER_TILE
    out_base = wid * OUT_PER_TILE

    pltpu.sync_copy(table_hbm, table_v)

    def round_body(g, carry):
        pltpu.sync_copy(
            idx_hbm.at[pl.ds(idx_base + g * IDX_PER_ROUND, IDX_PER_ROUND)],
            idx_v,
        )

        def chunk_body(c, carry2):
            iv = idx_v[pl.ds(c * LANES, LANES)]
            addr = iv * E
            p = c * LANES + lax.iota(jnp.int32, LANES)
            b = p // L
            posb = b * (E * L) + (p - b * L)
            for e in range(E):
                vals = plsc.load_gather(table_v, [addr + e])
                plsc.store_scatter(out_v, [posb + e * L], vals)
            return carry2

        lax.fori_loop(0, IDX_PER_ROUND // LANES, chunk_body, 0)

        pltpu.sync_copy(
            out_v,
            out_hbm.at[pl.ds(out_base + g * OUT_PER_ROUND, OUT_PER_ROUND)],
        )
        return carry

    lax.fori_loop(0, ROUNDS, round_body, 0)


def kernel(input_x, table):
    x = input_x.reshape(-1).astype(jnp.int32)
    table = table.astype(jnp.float32)

    table = table.reshape(-1)
    mesh = plsc.VectorSubcoreMesh(
        core_axis_name="c", subcore_axis_name="s",
        num_cores=NC, num_subcores=NS,
    )
    out = pl.kernel(
        _body,
        out_type=jax.ShapeDtypeStruct((B * E * L,), jnp.float32),
        mesh=mesh,
        compiler_params=pltpu.CompilerParams(needs_layout_passes=False),
        scratch_types=[
            pltpu.VMEM((V * E,), jnp.float32),
            pltpu.VMEM((IDX_PER_ROUND,), jnp.int32),
            pltpu.VMEM((OUT_PER_ROUND,), jnp.float32),
        ],
    )(x, table)
    return out.reshape(B, E, L)


# speedup vs baseline: 6.2319x; 1.8450x over previous
import jax
import jax.numpy as jnp
from jax import lax
from jax.experimental import pallas as pl
from jax.experimental.pallas import tpu as pltpu
from jax.experimental.pallas import tpu_sc as plsc

B = 16384
L = 200
E = 32
V = 257

NC = 2
NS = 16
NW = NC * NS
LANES = 16

BPT = B // NW
CB = 8
ROUNDS = BPT // CB
IPR = CB * L
OPR = CB * E * L
CHUNKS = IPR // LANES
IDX_PER_TILE = BPT * L
OUT_PER_TILE = BPT * E * L


def _body(idx_hbm, table_hbm, out_hbm, table_v, idx_v0, idx_v1,
          out_v0, out_v1, sem_in0, sem_in1, sem_out0, sem_out1):
    wid = lax.axis_index("s") * NC + lax.axis_index("c")
    idx_base = wid * IDX_PER_TILE
    out_base = wid * OUT_PER_TILE

    pltpu.sync_copy(table_hbm, table_v)

    pltpu.async_copy(idx_hbm.at[pl.ds(idx_base, IPR)], idx_v0, sem_in0)
    pltpu.async_copy(idx_hbm.at[pl.ds(idx_base + IPR, IPR)], idx_v1, sem_in1)

    def compute_round(ib, ob):
        def chunk_body(c, carry2):
            iv = ib[pl.ds(c * LANES, LANES)]
            p = c * LANES + lax.iota(jnp.int32, LANES)
            b = p // L
            posb = b * (E * L) + (p - b * L)
            for e in range(E):
                vals = plsc.load_gather(table_v, [iv + e * V])
                plsc.store_scatter(ob, [posb + e * L], vals)
            return carry2

        lax.fori_loop(0, CHUNKS, chunk_body, 0)

    def pair_body(i, carry):
        for s, ib, ob, sem_in, sem_out in (
                (0, idx_v0, out_v0, sem_in0, sem_out0),
                (1, idx_v1, out_v1, sem_in1, sem_out1)):
            g = 2 * i + s
            pltpu.make_async_copy(idx_hbm.at[pl.ds(0, IPR)], ib, sem_in).wait()

            @pl.when(g >= 2)
            def _():
                pltpu.make_async_copy(
                    ob, out_hbm.at[pl.ds(0, OPR)], sem_out).wait()

            compute_round(ib, ob)
            pltpu.async_copy(
                ob, out_hbm.at[pl.ds(out_base + g * OPR, OPR)], sem_out)

            @pl.when(g + 2 < ROUNDS)
            def _():
                g2 = jnp.minimum(g + 2, ROUNDS - 1)
                pltpu.async_copy(
                    idx_hbm.at[pl.ds(idx_base + g2 * IPR, IPR)], ib, sem_in)
        return carry

    lax.fori_loop(0, ROUNDS // 2, pair_body, 0)

    pltpu.make_async_copy(out_v0, out_hbm.at[pl.ds(0, OPR)], sem_out0).wait()
    pltpu.make_async_copy(out_v1, out_hbm.at[pl.ds(0, OPR)], sem_out1).wait()


def kernel(input_x, table):
    x = input_x.reshape(-1).astype(jnp.int32)
    table_t = table.astype(jnp.float32).T.reshape(-1)

    mesh = plsc.VectorSubcoreMesh(
        core_axis_name="c", subcore_axis_name="s",
        num_cores=NC, num_subcores=NS,
    )
    out = pl.kernel(
        _body,
        out_type=jax.ShapeDtypeStruct((B * E * L,), jnp.float32),
        mesh=mesh,
        compiler_params=pltpu.CompilerParams(needs_layout_passes=False),
        scratch_types=[
            pltpu.VMEM((E * V,), jnp.float32),
            pltpu.VMEM((IPR,), jnp.int32),
            pltpu.VMEM((IPR,), jnp.int32),
            pltpu.VMEM((OPR,), jnp.float32),
            pltpu.VMEM((OPR,), jnp.float32),
            pltpu.SemaphoreType.DMA,
            pltpu.SemaphoreType.DMA,
            pltpu.SemaphoreType.DMA,
            pltpu.SemaphoreType.DMA,
        ],
    )(x, table_t)
    return out.reshape(B, E, L)


# speedup vs baseline: 7.4023x; 1.1878x over previous
import jax
import jax.numpy as jnp
from jax import lax
from jax.experimental import pallas as pl
from jax.experimental.pallas import tpu as pltpu
from jax.experimental.pallas import tpu_sc as plsc

B = 16384
L = 200
E = 32
V = 257

NC = 2
NS = 16
NW = NC * NS
LANES = 16

BPT = B // NW
CB = 8
ROUNDS = BPT // CB
IPR = CB * L
OPR = CB * E * L
CHUNKS = IPR // LANES
IDX_PER_TILE = BPT * L
OUT_PER_TILE = BPT * E * L


def _body(idx_hbm, table_hbm, out_hbm, table_v, idx_v0, idx_v1,
          out_v0, out_v1, sem_in0, sem_in1, sem_out0, sem_out1):
    wid = lax.axis_index("s") * NC + lax.axis_index("c")
    idx_base = wid * IDX_PER_TILE
    out_base = wid * OUT_PER_TILE

    pltpu.sync_copy(table_hbm, table_v)

    pltpu.async_copy(idx_hbm.at[pl.ds(idx_base, IPR)], idx_v0, sem_in0)
    pltpu.async_copy(idx_hbm.at[pl.ds(idx_base + IPR, IPR)], idx_v1, sem_in1)

    def compute_round(ib, ob):
        @plsc.parallel_loop(0, CHUNKS, 1, unroll=4)
        def _(c):
            iv = ib[pl.ds(c * LANES, LANES)]
            p = c * LANES + lax.iota(jnp.int32, LANES)
            b = p // L
            posb = b * (E * L) + (p - b * L)
            for e in range(E):
                vals = plsc.load_gather(table_v, [iv + e * V])
                plsc.store_scatter(ob, [posb + e * L], vals)

    def pair_body(i, carry):
        for s, ib, ob, sem_in, sem_out in (
                (0, idx_v0, out_v0, sem_in0, sem_out0),
                (1, idx_v1, out_v1, sem_in1, sem_out1)):
            g = 2 * i + s
            pltpu.make_async_copy(idx_hbm.at[pl.ds(0, IPR)], ib, sem_in).wait()

            @pl.when(g >= 2)
            def _():
                pltpu.make_async_copy(
                    ob, out_hbm.at[pl.ds(0, OPR)], sem_out).wait()

            compute_round(ib, ob)
            pltpu.async_copy(
                ob, out_hbm.at[pl.ds(out_base + g * OPR, OPR)], sem_out)

            @pl.when(g + 2 < ROUNDS)
            def _():
                g2 = jnp.minimum(g + 2, ROUNDS - 1)
                pltpu.async_copy(
                    idx_hbm.at[pl.ds(idx_base + g2 * IPR, IPR)], ib, sem_in)
        return carry

    lax.fori_loop(0, ROUNDS // 2, pair_body, 0)

    pltpu.make_async_copy(out_v0, out_hbm.at[pl.ds(0, OPR)], sem_out0).wait()
    pltpu.make_async_copy(out_v1, out_hbm.at[pl.ds(0, OPR)], sem_out1).wait()


def kernel(input_x, table):
    x = input_x.reshape(-1).astype(jnp.int32)
    table_t = table.astype(jnp.float32).T.reshape(-1)

    mesh = plsc.VectorSubcoreMesh(
        core_axis_name="c", subcore_axis_name="s",
        num_cores=NC, num_subcores=NS,
    )
    out = pl.kernel(
        _body,
        out_type=jax.ShapeDtypeStruct((B * E * L,), jnp.float32),
        mesh=mesh,
        compiler_params=pltpu.CompilerParams(needs_layout_passes=False),
        scratch_types=[
            pltpu.VMEM((E * V,), jnp.float32),
            pltpu.VMEM((IPR,), jnp.int32),
            pltpu.VMEM((IPR,), jnp.int32),
            pltpu.VMEM((OPR,), jnp.float32),
            pltpu.VMEM((OPR,), jnp.float32),
            pltpu.SemaphoreType.DMA,
            pltpu.SemaphoreType.DMA,
            pltpu.SemaphoreType.DMA,
            pltpu.SemaphoreType.DMA,
        ],
    )(x, table_t)
    return out.reshape(B, E, L)


# speedup vs baseline: 12.3337x; 1.6662x over previous
import jax
import jax.numpy as jnp
from jax import lax
from jax.experimental import pallas as pl
from jax.experimental.pallas import tpu as pltpu
from jax.experimental.pallas import tpu_sc as plsc

B = 16384
L = 200
E = 32
V = 257

NC = 2
NS = 16
NW = NC * NS
LANES = 16

BPT = B // NW
CB = 4
ROUNDS = BPT // CB
IPR = CB * L
OPR = CB * E * L
CHUNKS = IPR // LANES
IDX_PER_TILE = BPT * L
OUT_PER_TILE = BPT * E * L


def _body(idx_hbm, table_hbm, out_hbm, table_v, idx_v0, idx_v1,
          out_v0, out_v1, sem_in0, sem_in1, sem_out0, sem_out1):
    wid = lax.axis_index("s") * NC + lax.axis_index("c")
    idx_base = wid * IDX_PER_TILE
    out_base = wid * BPT

    pltpu.sync_copy(table_hbm, table_v)

    pltpu.async_copy(idx_hbm.at[pl.ds(idx_base, IPR)], idx_v0, sem_in0)
    pltpu.async_copy(idx_hbm.at[pl.ds(idx_base + IPR, IPR)], idx_v1, sem_in1)

    def compute_round(ib, ob):
        @plsc.parallel_loop(0, CHUNKS, 1, unroll=4)
        def _(c):
            iv = ib[pl.ds(c * LANES, LANES)]
            p = c * LANES + lax.iota(jnp.int32, LANES)
            b = p // L
            lpos = p - b * L
            for e in range(E):
                vals = plsc.load_gather(table_v, [iv + e * V])
                plsc.store_scatter(
                    ob, [b, jnp.full((LANES,), e, jnp.int32), lpos], vals)

    def pair_body(i, carry):
        for s, ib, ob, sem_in, sem_out in (
                (0, idx_v0, out_v0, sem_in0, sem_out0),
                (1, idx_v1, out_v1, sem_in1, sem_out1)):
            g = 2 * i + s
            pltpu.make_async_copy(idx_hbm.at[pl.ds(0, IPR)], ib, sem_in).wait()

            @pl.when(g >= 2)
            def _():
                pltpu.make_async_copy(
                    ob, out_hbm.at[pl.ds(0, CB)], sem_out).wait()

            compute_round(ib, ob)
            pltpu.async_copy(
                ob, out_hbm.at[pl.ds(out_base + g * CB, CB)], sem_out)

            @pl.when(g + 2 < ROUNDS)
            def _():
                g2 = jnp.minimum(g + 2, ROUNDS - 1)
                pltpu.async_copy(
                    idx_hbm.at[pl.ds(idx_base + g2 * IPR, IPR)], ib, sem_in)
        return carry

    lax.fori_loop(0, ROUNDS // 2, pair_body, 0)

    pltpu.make_async_copy(out_v0, out_hbm.at[pl.ds(0, CB)], sem_out0).wait()
    pltpu.make_async_copy(out_v1, out_hbm.at[pl.ds(0, CB)], sem_out1).wait()


def kernel(input_x, table):
    x = input_x.reshape(-1).astype(jnp.int32)
    table_t = table.astype(jnp.float32).T.reshape(-1)

    mesh = plsc.VectorSubcoreMesh(
        core_axis_name="c", subcore_axis_name="s",
        num_cores=NC, num_subcores=NS,
    )
    out = pl.kernel(
        _body,
        out_type=jax.ShapeDtypeStruct((B, E, L), jnp.float32),
        mesh=mesh,
        compiler_params=pltpu.CompilerParams(needs_layout_passes=False),
        scratch_types=[
            pltpu.VMEM((E * V,), jnp.float32),
            pltpu.VMEM((IPR,), jnp.int32),
            pltpu.VMEM((IPR,), jnp.int32),
            pltpu.VMEM((CB, E, L), jnp.float32),
            pltpu.VMEM((CB, E, L), jnp.float32),
            pltpu.SemaphoreType.DMA,
            pltpu.SemaphoreType.DMA,
            pltpu.SemaphoreType.DMA,
            pltpu.SemaphoreType.DMA,
        ],
    )(x, table_t)
    return out
